# baseline (speedup 1.0000x reference)
import jax
import jax.numpy as jnp
from jax.experimental import pallas as pl
from jax.experimental.pallas import tpu as pltpu

S, B, H, V = 8, 16, 2048, 1000
SB = S * B
G = 3 * H

KC = 256
NK = H // KC
EC = 200
NE = V // EC
OC = 256
NO = H // OC


def _sub_copies(hbm, buf, sem, row0, rows, nsub):
    sub = rows // nsub
    return [
        pltpu.make_async_copy(
            hbm.at[pl.ds(row0 + s * sub, sub), :],
            buf.at[pl.ds(s * sub, sub), :],
            sem,
        )
        for s in range(nsub)
    ]


def _issue(hbm, buf, sem, row0, rows, nsub):
    for c in _sub_copies(hbm, buf, sem, row0, rows, nsub):
        c.start()


def _wait(hbm, buf, sem, row0, rows, nsub):
    for c in _sub_copies(hbm, buf, sem, row0, rows, nsub):
        c.wait()


def _mega_body(seq_ref, inp_ref, binit_ref, bih_ref, bhh_ref, bout_ref,
               emb_hbm, wi_hbm, wih_hbm, whh_hbm, wo_hbm,
               lp_ref, hs_ref,
               whh_bf, blk, wiblk, embblk, woblk, gi, x_buf, h_buf,
               sem_blk, sem_wi, sem_emb, sem_wo):
    f32 = jnp.float32

    def blk_src(j):
        return (wih_hbm, j * KC) if j < NK else (whh_hbm, (j - NK) * KC)

    def blk_issue(j):
        src, r0 = blk_src(j)
        _issue(src, blk.at[j % 3], sem_blk.at[j % 3], r0, KC, 4)

    def blk_wait(j):
        src, r0 = blk_src(j)
        _wait(src, blk.at[j % 3], sem_blk.at[j % 3], r0, KC, 4)

    _issue(wi_hbm, wiblk.at[0], sem_wi.at[0], 0, KC, 4)
    _issue(emb_hbm, embblk.at[0], sem_emb.at[0], 0, EC, 5)
    blk_issue(0)
    blk_issue(1)

    h_buf[...] = jnp.broadcast_to(binit_ref[...], (B, H))
    for k in range(NK):
        if k + 1 < NK:
            _issue(wi_hbm, wiblk.at[(k + 1) % 2], sem_wi.at[(k + 1) % 2],
                   (k + 1) * KC, KC, 4)
        _wait(wi_hbm, wiblk.at[k % 2], sem_wi.at[k % 2], k * KC, KC, 4)
        h_buf[...] += jnp.dot(inp_ref[:, k * KC:(k + 1) * KC],
                              wiblk[k % 2], preferred_element_type=f32)
    h_buf[...] = jnp.tanh(h_buf[...])

    ids = seq_ref[...]
    base_iota = jax.lax.broadcasted_iota(jnp.int32, (SB, EC), 1)
    for j in range(NE):
        if j + 1 < NE:
            _issue(emb_hbm, embblk.at[(j + 1) % 2], sem_emb.at[(j + 1) % 2],
                   (j + 1) * EC, EC, 5)
        _wait(emb_hbm, embblk.at[j % 2], sem_emb.at[j % 2], j * EC, EC, 5)
        onehot = (ids == base_iota + j * EC).astype(f32)
        part = jnp.dot(onehot, embblk[j % 2], preferred_element_type=f32)
        if j == 0:
            x_buf[...] = part
        else:
            x_buf[...] += part

    gi[...] = jnp.broadcast_to(bih_ref[...], (SB, G))
    for j in range(2 * NK):
        if j + 2 < 2 * NK:
            blk_issue(j + 2)
        blk_wait(j)
        if j < NK:
            gi[...] += jnp.dot(x_buf[:, j * KC:(j + 1) * KC],
                               blk[j % 3], preferred_element_type=f32)
        else:
            k = j - NK
            whh_bf[pl.ds(k * KC, KC), :] = blk[j % 3].astype(jnp.bfloat16)

    _issue(wo_hbm, woblk.at[0], sem_wo.at[0], 0, OC, 2)
    _issue(wo_hbm, woblk.at[1], sem_wo.at[1], OC, OC, 2)

    def step(t, h):
        gh = jnp.dot(h.astype(jnp.bfloat16), whh_bf[...],
                     preferred_element_type=f32) + bhh_ref[...]
        gi_t = gi[pl.ds(t * B, B), :]
        r = jax.nn.sigmoid(gi_t[:, 0:H] + gh[:, 0:H])
        z = jax.nn.sigmoid(gi_t[:, H:2 * H] + gh[:, H:2 * H])
        n = jnp.tanh(gi_t[:, 2 * H:3 * H] + r * gh[:, 2 * H:3 * H])
        h_new = (1.0 - z) * n + z * h
        hs_ref[pl.ds(t * B, B), :] = h_new
        return h_new

    jax.lax.fori_loop(0, S, step, h_buf[...])

    lp_ref[...] = jnp.broadcast_to(bout_ref[...], (SB, V))
    for k in range(NO):
        _wait(wo_hbm, woblk.at[k % 2], sem_wo.at[k % 2], k * OC, OC, 2)
        lp_ref[...] += jnp.dot(hs_ref[:, k * OC:(k + 1) * OC],
                               woblk[k % 2], preferred_element_type=f32)
        if k + 2 < NO:
            _issue(wo_hbm, woblk.at[k % 2], sem_wo.at[k % 2],
                   (k + 2) * OC, OC, 2)
    logits = lp_ref[...]
    m = jnp.max(logits, axis=-1, keepdims=True)
    shifted = logits - m
    lse = jnp.log(jnp.sum(jnp.exp(shifted), axis=-1, keepdims=True))
    lp_ref[...] = shifted - lse


def kernel(seq_part, seq_length, input, emb, W_init, b_init, W_ih, W_hh, b_ih, b_hh, W_out, b_out):
    del seq_length
    seq2d = seq_part.reshape(SB, 1)

    vmem = pl.BlockSpec(memory_space=pltpu.VMEM)
    hbm = pl.BlockSpec(memory_space=pl.ANY)

    log_probs, hs = pl.pallas_call(
        _mega_body,
        in_specs=[vmem] * 6 + [hbm] * 5,
        out_specs=(vmem, vmem),
        out_shape=(
            jax.ShapeDtypeStruct((SB, V), jnp.float32),
            jax.ShapeDtypeStruct((SB, H), jnp.float32),
        ),
        scratch_shapes=[
            pltpu.VMEM((H, G), jnp.bfloat16),
            pltpu.VMEM((3, KC, G), jnp.float32),
            pltpu.VMEM((2, KC, H), jnp.float32),
            pltpu.VMEM((2, EC, H), jnp.float32),
            pltpu.VMEM((2, OC, V), jnp.float32),
            pltpu.VMEM((SB, G), jnp.float32),
            pltpu.VMEM((SB, H), jnp.float32),
            pltpu.VMEM((B, H), jnp.float32),
            pltpu.SemaphoreType.DMA((3,)),
            pltpu.SemaphoreType.DMA((2,)),
            pltpu.SemaphoreType.DMA((2,)),
            pltpu.SemaphoreType.DMA((2,)),
        ],
    )(seq2d, input, b_init.reshape(1, H), b_ih.reshape(1, G),
      b_hh.reshape(1, G), b_out.reshape(1, V),
      emb, W_init, W_ih, W_hh, W_out)

    hidden = hs[(S - 1) * B:].reshape(1, B, H)
    return log_probs.reshape(S, B, V), hidden


# speedup vs baseline: 1.0596x; 1.0596x over previous
import jax
import jax.numpy as jnp
from jax.experimental import pallas as pl
from jax.experimental.pallas import tpu as pltpu

S, B, H, V = 8, 16, 2048, 1000
SB = S * B
G = 3 * H

KC = 256
NK = H // KC
OC = 256
NO = H // OC


def _sub_copies(hbm, buf, sem, row0, rows, nsub):
    sub = rows // nsub
    return [
        pltpu.make_async_copy(
            hbm.at[pl.ds(row0 + s * sub, sub), :],
            buf.at[pl.ds(s * sub, sub), :],
            sem,
        )
        for s in range(nsub)
    ]


def _issue(hbm, buf, sem, row0, rows, nsub):
    for c in _sub_copies(hbm, buf, sem, row0, rows, nsub):
        c.start()


def _wait(hbm, buf, sem, row0, rows, nsub):
    for c in _sub_copies(hbm, buf, sem, row0, rows, nsub):
        c.wait()


def _mega_body(seq_ref, inp_ref, binit_ref, bih_ref, bhh_ref, bout_ref,
               emb_hbm, wi_hbm, wih_hbm, whh_hbm, wo_hbm,
               lp_ref, hs_ref,
               whh_bf, blk, wiblk, woblk, gi, x_buf, h_buf,
               sem_blk, sem_wi, sem_emb, sem_wo):
    f32 = jnp.float32

    def gather_copy(i):
        return pltpu.make_async_copy(
            emb_hbm.at[pl.ds(seq_ref[i], 1), :],
            x_buf.at[pl.ds(i, 1), :],
            sem_emb,
        )

    def blk_src(j):
        return (wih_hbm, j * KC) if j < NK else (whh_hbm, (j - NK) * KC)

    def blk_issue(j):
        src, r0 = blk_src(j)
        _issue(src, blk.at[j % 2], sem_blk.at[j % 2], r0, KC, 8)

    def blk_wait(j):
        src, r0 = blk_src(j)
        _wait(src, blk.at[j % 2], sem_blk.at[j % 2], r0, KC, 8)

    _issue(wi_hbm, wiblk.at[0], sem_wi.at[0], 0, KC, 4)
    for i in range(SB):
        gather_copy(i).start()
    blk_issue(0)

    h_buf[...] = jnp.broadcast_to(binit_ref[...], (B, H))
    for k in range(NK):
        if k + 1 < NK:
            _issue(wi_hbm, wiblk.at[(k + 1) % 2], sem_wi.at[(k + 1) % 2],
                   (k + 1) * KC, KC, 4)
        _wait(wi_hbm, wiblk.at[k % 2], sem_wi.at[k % 2], k * KC, KC, 4)
        h_buf[...] += jnp.dot(inp_ref[:, k * KC:(k + 1) * KC],
                              wiblk[k % 2], preferred_element_type=f32)
    h_buf[...] = jnp.tanh(h_buf[...])

    for i in range(SB):
        gather_copy(i).wait()

    gi[...] = jnp.broadcast_to(bih_ref[...], (SB, G))
    for j in range(2 * NK):
        if j + 1 < 2 * NK:
            blk_issue(j + 1)
        blk_wait(j)
        if j < NK:
            gi[...] += jnp.dot(x_buf[:, j * KC:(j + 1) * KC],
                               blk[j % 2], preferred_element_type=f32)
        else:
            k = j - NK
            whh_bf[pl.ds(k * KC, KC), :] = blk[j % 2].astype(jnp.bfloat16)

    _issue(wo_hbm, woblk.at[0], sem_wo.at[0], 0, OC, 2)
    _issue(wo_hbm, woblk.at[1], sem_wo.at[1], OC, OC, 2)

    def step(t, h):
        gh = jnp.dot(h.astype(jnp.bfloat16), whh_bf[...],
                     preferred_element_type=f32) + bhh_ref[...]
        gi_t = gi[pl.ds(t * B, B), :]
        r = jax.nn.sigmoid(gi_t[:, 0:H] + gh[:, 0:H])
        z = jax.nn.sigmoid(gi_t[:, H:2 * H] + gh[:, H:2 * H])
        n = jnp.tanh(gi_t[:, 2 * H:3 * H] + r * gh[:, 2 * H:3 * H])
        h_new = (1.0 - z) * n + z * h
        hs_ref[pl.ds(t * B, B), :] = h_new
        return h_new

    jax.lax.fori_loop(0, S, step, h_buf[...])

    lp_ref[...] = jnp.broadcast_to(bout_ref[...], (SB, V))
    for k in range(NO):
        _wait(wo_hbm, woblk.at[k % 2], sem_wo.at[k % 2], k * OC, OC, 2)
        lp_ref[...] += jnp.dot(hs_ref[:, k * OC:(k + 1) * OC],
                               woblk[k % 2], preferred_element_type=f32)
        if k + 2 < NO:
            _issue(wo_hbm, woblk.at[k % 2], sem_wo.at[k % 2],
                   (k + 2) * OC, OC, 2)
    logits = lp_ref[...]
    m = jnp.max(logits, axis=-1, keepdims=True)
    shifted = logits - m
    lse = jnp.log(jnp.sum(jnp.exp(shifted), axis=-1, keepdims=True))
    lp_ref[...] = shifted - lse


def kernel(seq_part, seq_length, input, emb, W_init, b_init, W_ih, W_hh, b_ih, b_hh, W_out, b_out):
    del seq_length
    seq1d = seq_part.reshape(SB)

    vmem = pl.BlockSpec(memory_space=pltpu.VMEM)
    smem = pl.BlockSpec(memory_space=pltpu.SMEM)
    hbm = pl.BlockSpec(memory_space=pl.ANY)

    log_probs, hs = pl.pallas_call(
        _mega_body,
        in_specs=[smem] + [vmem] * 5 + [hbm] * 5,
        out_specs=(vmem, vmem),
        out_shape=(
            jax.ShapeDtypeStruct((SB, V), jnp.float32),
            jax.ShapeDtypeStruct((SB, H), jnp.float32),
        ),
        scratch_shapes=[
            pltpu.VMEM((H, G), jnp.bfloat16),
            pltpu.VMEM((2, KC, G), jnp.float32),
            pltpu.VMEM((2, KC, H), jnp.float32),
            pltpu.VMEM((2, OC, V), jnp.float32),
            pltpu.VMEM((SB, G), jnp.float32),
            pltpu.VMEM((SB, H), jnp.float32),
            pltpu.VMEM((B, H), jnp.float32),
            pltpu.SemaphoreType.DMA((2,)),
            pltpu.SemaphoreType.DMA((2,)),
            pltpu.SemaphoreType.DMA,
            pltpu.SemaphoreType.DMA((2,)),
        ],
    )(seq1d, input, b_init.reshape(1, H), b_ih.reshape(1, G),
      b_hh.reshape(1, G), b_out.reshape(1, V),
      emb, W_init, W_ih, W_hh, W_out)

    hidden = hs[(S - 1) * B:].reshape(1, B, H)
    return log_probs.reshape(S, B, V), hidden


# speedup vs baseline: 1.1066x; 1.0443x over previous
import jax
import jax.numpy as jnp
from jax.experimental import pallas as pl
from jax.experimental.pallas import tpu as pltpu

S, B, H, V = 8, 16, 2048, 1000
SB = S * B
G = 3 * H

KC = 256
NK = H // KC
OC = 1024
NO = H // OC


def _sub_copies(hbm, buf, sem, row0, rows, nsub):
    sub = rows // nsub
    return [
        pltpu.make_async_copy(
            hbm.at[pl.ds(row0 + s * sub, sub), :],
            buf.at[pl.ds(s * sub, sub), :],
            sem,
        )
        for s in range(nsub)
    ]


def _issue(hbm, buf, sem, row0, rows, nsub):
    for c in _sub_copies(hbm, buf, sem, row0, rows, nsub):
        c.start()


def _wait(hbm, buf, sem, row0, rows, nsub):
    for c in _sub_copies(hbm, buf, sem, row0, rows, nsub):
        c.wait()


def _mega_body(seq_ref, inp_ref, binit_ref, bih_ref, bhh_ref, bout_ref,
               emb_hbm, wi_hbm, wih_hbm, whh_hbm, wo_hbm,
               lp_ref, hs_ref,
               whh_bf, blk, wiblk, woblk, gi, x_buf, h_buf,
               sem_blk, sem_wi, sem_emb, sem_wo):
    f32 = jnp.float32

    def gather_copy(i):
        return pltpu.make_async_copy(
            emb_hbm.at[pl.ds(seq_ref[i], 1), :],
            x_buf.at[pl.ds(i, 1), :],
            sem_emb,
        )

    def blk_src(j):
        return (wih_hbm, j * KC) if j < NK else (whh_hbm, (j - NK) * KC)

    def blk_issue(j):
        src, r0 = blk_src(j)
        _issue(src, blk.at[j % 2], sem_blk.at[j % 2], r0, KC, 8)

    def blk_wait(j):
        src, r0 = blk_src(j)
        _wait(src, blk.at[j % 2], sem_blk.at[j % 2], r0, KC, 8)

    _issue(wi_hbm, wiblk.at[0], sem_wi.at[0], 0, KC, 4)
    for i in range(SB):
        gather_copy(i).start()
    blk_issue(0)

    h_buf[...] = jnp.broadcast_to(binit_ref[...], (B, H))
    for k in range(NK):
        if k + 1 < NK:
            _issue(wi_hbm, wiblk.at[(k + 1) % 2], sem_wi.at[(k + 1) % 2],
                   (k + 1) * KC, KC, 4)
        _wait(wi_hbm, wiblk.at[k % 2], sem_wi.at[k % 2], k * KC, KC, 4)
        h_buf[...] += jnp.dot(inp_ref[:, k * KC:(k + 1) * KC],
                              wiblk[k % 2], preferred_element_type=f32)
    h_buf[...] = jnp.tanh(h_buf[...])

    for i in range(SB):
        gather_copy(i).wait()

    gi[...] = jnp.broadcast_to(bih_ref[...], (SB, G))
    for j in range(2 * NK):
        if j + 1 < 2 * NK:
            blk_issue(j + 1)
        blk_wait(j)
        if j < NK:
            gi[...] += jnp.dot(x_buf[:, j * KC:(j + 1) * KC],
                               blk[j % 2], preferred_element_type=f32)
        else:
            k = j - NK
            whh_bf[pl.ds(k * KC, KC), :] = blk[j % 2].astype(jnp.bfloat16)

    _issue(wo_hbm, woblk.at[0], sem_wo.at[0], 0, OC, 4)
    _issue(wo_hbm, woblk.at[1], sem_wo.at[1], OC, OC, 4)

    def step(t, h):
        gh = jnp.dot(h.astype(jnp.bfloat16), whh_bf[...],
                     preferred_element_type=f32) + bhh_ref[...]
        gi_t = gi[pl.ds(t * B, B), :]
        r = jax.nn.sigmoid(gi_t[:, 0:H] + gh[:, 0:H])
        z = jax.nn.sigmoid(gi_t[:, H:2 * H] + gh[:, H:2 * H])
        n = jnp.tanh(gi_t[:, 2 * H:3 * H] + r * gh[:, 2 * H:3 * H])
        h_new = (1.0 - z) * n + z * h
        hs_ref[pl.ds(t * B, B), :] = h_new
        return h_new

    jax.lax.fori_loop(0, S, step, h_buf[...])

    lp_ref[...] = jnp.broadcast_to(bout_ref[...], (SB, V))
    for k in range(NO):
        _wait(wo_hbm, woblk.at[k % 2], sem_wo.at[k % 2], k * OC, OC, 4)
        lp_ref[...] += jnp.dot(hs_ref[:, k * OC:(k + 1) * OC],
                               woblk[k % 2], preferred_element_type=f32)
    logits = lp_ref[...]
    m = jnp.max(logits, axis=-1, keepdims=True)
    shifted = logits - m
    lse = jnp.log(jnp.sum(jnp.exp(shifted), axis=-1, keepdims=True))
    lp_ref[...] = shifted - lse


def kernel(seq_part, seq_length, input, emb, W_init, b_init, W_ih, W_hh, b_ih, b_hh, W_out, b_out):
    del seq_length
    seq1d = seq_part.reshape(SB)

    vmem = pl.BlockSpec(memory_space=pltpu.VMEM)
    smem = pl.BlockSpec(memory_space=pltpu.SMEM)
    hbm = pl.BlockSpec(memory_space=pl.ANY)

    log_probs, hs = pl.pallas_call(
        _mega_body,
        in_specs=[smem] + [vmem] * 5 + [hbm] * 5,
        out_specs=(vmem, vmem),
        out_shape=(
            jax.ShapeDtypeStruct((SB, V), jnp.float32),
            jax.ShapeDtypeStruct((SB, H), jnp.float32),
        ),
        scratch_shapes=[
            pltpu.VMEM((H, G), jnp.bfloat16),
            pltpu.VMEM((2, KC, G), jnp.float32),
            pltpu.VMEM((2, KC, H), jnp.float32),
            pltpu.VMEM((2, OC, V), jnp.float32),
            pltpu.VMEM((SB, G), jnp.float32),
            pltpu.VMEM((SB, H), jnp.float32),
            pltpu.VMEM((B, H), jnp.float32),
            pltpu.SemaphoreType.DMA((2,)),
            pltpu.SemaphoreType.DMA((2,)),
            pltpu.SemaphoreType.DMA,
            pltpu.SemaphoreType.DMA((2,)),
        ],
    )(seq1d, input, b_init.reshape(1, H), b_ih.reshape(1, G),
      b_hh.reshape(1, G), b_out.reshape(1, V),
      emb, W_init, W_ih, W_hh, W_out)

    hidden = hs[(S - 1) * B:].reshape(1, B, H)
    return log_probs.reshape(S, B, V), hidden
